# Initial kernel scaffold; baseline (speedup 1.0000x reference)
import jax
import jax.numpy as jnp
from jax.experimental import pallas as pl


def kernel(h, edge_index, graph_ids, W1, b1, W2, b2, Wc, bc):
    raise NotImplementedError("write your pallas kernel here")


# speedup vs baseline: 4.4364x; 4.4364x over previous
import functools

import jax
import jax.numpy as jnp
from jax import lax
from jax.experimental import pallas as pl
from jax.experimental.pallas import tpu as pltpu
from jax.experimental.pallas import tpu_sc as plsc

N = 10000
E = 320000
D = 128
H = 256
C = 32
G = 64

NC, NS, LANES = 2, 16, 16
NW = NC * NS
NPAD = 10240
RPT = NPAD // NS
EK = 80

_mesh = plsc.VectorSubcoreMesh(
    core_axis_name="c", subcore_axis_name="s", num_cores=NC, num_subcores=NS)


DK = 2000
D_EPT = E // NS


@functools.partial(
    pl.kernel,
    out_type=jax.ShapeDtypeStruct((NC, NS, N), jnp.int32),
    mesh=_mesh,
    scratch_types=[
        pltpu.VMEM((DK,), jnp.int32),
        pltpu.VMEM((N,), jnp.int32),
    ],
    compiler_params=pltpu.CompilerParams(needs_layout_passes=False),
)
def _degrees(src_hbm, dst_hbm, out_hbm, idx_v, hist_v):
    c = lax.axis_index("c")
    s = lax.axis_index("s")
    zv = jnp.zeros((LANES,), jnp.int32)

    @pl.loop(0, N // LANES)
    def _(i):
        hist_v[pl.ds(i * LANES, LANES)] = zv

    @pl.loop(0, D_EPT // DK)
    def _(i):
        base = s * D_EPT + i * DK

        @pl.when(c == 0)
        def _():
            pltpu.sync_copy(src_hbm.at[pl.ds(base, DK)], idx_v)

        @pl.when(c == 1)
        def _():
            pltpu.sync_copy(dst_hbm.at[pl.ds(base, DK)], idx_v)

        ones = jnp.ones((LANES,), jnp.int32)

        @pl.loop(0, DK // LANES)
        def _(j):
            idx = idx_v[pl.ds(j * LANES, LANES)]
            plsc.addupdate_scatter(hist_v, [idx], ones)

    pltpu.sync_copy(hist_v, out_hbm.at[c, s])


def _norm_body(degp_ref, norm_ref):
    d = jnp.sum(degp_ref[...].astype(jnp.float32), axis=1)
    norm_ref[...] = jnp.where(d > 0, lax.rsqrt(jnp.maximum(d, 1.0)), 0.0)


_norms = pl.pallas_call(
    _norm_body,
    out_shape=jax.ShapeDtypeStruct((NC, N), jnp.float32),
)


def _make_spmm(feat_split):
    ept = E // NS if feat_split else E // NW
    niter = ept // EK
    nsrc_rows = 2 * N if feat_split else N

    @functools.partial(
        pl.kernel,
        out_type=jax.ShapeDtypeStruct((NC, NPAD, 128), jnp.float32),
        mesh=_mesh,
        scratch_types=[
            pltpu.VMEM((EK,), jnp.int32),
            pltpu.VMEM((EK,), jnp.int32),
            pltpu.VMEM((EK, 128), jnp.float32),
            pltpu.VMEM((64, 128), jnp.float32),
            pltpu.VMEM_SHARED((NPAD, 128), jnp.float32),
            pltpu.SemaphoreType.DMA,
        ],
    )
    def spmm(xs_hbm, src_hbm, dst_hbm, out_hbm,
             sidx_v, didx_v, rows_v, zero_v, acc_sh, sem):
        c = lax.axis_index("c")
        s = lax.axis_index("s")
        zv = jnp.zeros((LANES,), jnp.float32)

        @pl.loop(0, 64)
        def _(j):
            for k in range(128 // LANES):
                zero_v[j, pl.ds(k * LANES, LANES)] = zv

        @pl.loop(0, RPT // 64)
        def _(t):
            pltpu.sync_copy(zero_v, acc_sh.at[pl.ds(s * RPT + t * 64, 64)])

        plsc.subcore_barrier()
        if feat_split:
            ebase = s * ept
            coff = c * N
        else:
            ebase = (s * NC + c) * ept
            coff = 0

        @pl.loop(0, niter)
        def _(i):
            base = ebase + i * EK
            pltpu.sync_copy(src_hbm.at[pl.ds(base, EK)], sidx_v)
            pltpu.sync_copy(dst_hbm.at[pl.ds(base, EK)], didx_v)
            if feat_split:
                for j in range(EK // LANES):
                    sl = pl.ds(j * LANES, LANES)
                    sidx_v[sl] = sidx_v[sl] + coff
            pltpu.async_copy(xs_hbm.at[sidx_v], rows_v, sem).wait()
            pltpu.sync_copy(rows_v, acc_sh.at[didx_v], add=True)

        plsc.subcore_barrier()
        pltpu.sync_copy(acc_sh.at[pl.ds(s * RPT, RPT)],
                        out_hbm.at[c, pl.ds(s * RPT, RPT)])

    assert nsrc_rows in (N, 2 * N)
    return spmm


_spmm_edges = _make_spmm(False)
_spmm_feats = _make_spmm(True)


GB = 16
NGRP = N // GB
GPW = 20
WROWS = GPW * GB
GH = G * H


@functools.partial(
    pl.kernel,
    out_type=jax.ShapeDtypeStruct((NW, GH), jnp.float32),
    mesh=_mesh,
    scratch_types=[
        pltpu.VMEM((WROWS,), jnp.int32),
        pltpu.VMEM((GB, H), jnp.float32),
        pltpu.VMEM((GH,), jnp.float32),
    ],
)
def _segmax(x_hbm, gid_hbm, out_hbm, gid_v, rows_v, acc_v):
    c = lax.axis_index("c")
    s = lax.axis_index("s")
    w = s * NC + c
    ninf = jnp.full((LANES,), -jnp.inf, jnp.float32)

    @pl.loop(0, GH // LANES)
    def _(i):
        acc_v[pl.ds(i * LANES, LANES)] = ninf

    pltpu.sync_copy(gid_hbm.at[pl.ds(w * WROWS, WROWS)], gid_v)
    ngroups = jnp.minimum(GPW, NGRP - w * GPW)

    def group_body(g, carry):
        prev_gid = carry[0]
        regs = list(carry[1:])
        r0 = w * WROWS + g * GB
        pltpu.sync_copy(x_hbm.at[pl.ds(r0, GB)], rows_v)
        gv = gid_v[pl.ds(g * GB, GB)]
        for jj in range(GB):
            gid_j = gv[jj]
            changed = gid_j != prev_gid

            @pl.when(changed & (prev_gid >= 0))
            def _(prev_gid=prev_gid, regs=tuple(regs)):
                for ch in range(H // LANES):
                    acc_v[pl.ds(prev_gid * H + ch * LANES, LANES)] = regs[ch]

            regs = [
                jnp.maximum(jnp.where(changed, ninf, r),
                            rows_v[jj, pl.ds(ch * LANES, LANES)])
                for ch, r in enumerate(regs)
            ]
            prev_gid = gid_j
        return (prev_gid, *regs)

    final = lax.fori_loop(0, ngroups, group_body,
                          (jnp.int32(-1), *([ninf] * (H // LANES))))
    prev_gid = final[0]
    regs = final[1:]

    @pl.when(prev_gid >= 0)
    def _():
        for ch in range(H // LANES):
            acc_v[pl.ds(prev_gid * H + ch * LANES, LANES)] = regs[ch]

    pltpu.sync_copy(acc_v, out_hbm.at[w])


BN = 2000


def _prep_body(h_ref, nsrc_ref, xs_ref):
    xs_ref[...] = h_ref[...] * nsrc_ref[...]


_prep = pl.pallas_call(
    _prep_body,
    grid=(N // BN,),
    in_specs=[
        pl.BlockSpec((BN, D), lambda i: (i, 0)),
        pl.BlockSpec((BN, 1), lambda i: (i, 0)),
    ],
    out_specs=pl.BlockSpec((BN, D), lambda i: (i, 0)),
    out_shape=jax.ShapeDtypeStruct((N, D), jnp.float32),
)


def _make_layer(din, scale_out):
    sum_parts = din == 128

    def body(*refs):
        if scale_out:
            agg_ref, ndst_ref, w_ref, b_ref, nsrc_ref, out_ref = refs
        else:
            agg_ref, ndst_ref, w_ref, b_ref, out_ref = refs
        if sum_parts:
            x = agg_ref[0] + agg_ref[1]
        else:
            x = jnp.concatenate([agg_ref[0], agg_ref[1]], axis=1)
        x = x * ndst_ref[...]
        y = jnp.dot(x, w_ref[...], preferred_element_type=jnp.float32)
        y = jnp.maximum(y + b_ref[...], 0.0)
        if scale_out:
            y = y * nsrc_ref[...]
            out_ref[0] = y[:, :H // 2]
            out_ref[1] = y[:, H // 2:]
        else:
            out_ref[...] = y

    in_specs = [
        pl.BlockSpec((2, BN, 128), lambda i: (0, i, 0)),
        pl.BlockSpec((BN, 1), lambda i: (i, 0)),
        pl.BlockSpec((din, H), lambda i: (0, 0)),
        pl.BlockSpec((1, H), lambda i: (0, 0)),
    ]
    if scale_out:
        in_specs.append(pl.BlockSpec((BN, 1), lambda i: (i, 0)))
        out_spec = pl.BlockSpec((2, BN, H // 2), lambda i: (0, i, 0))
        out_shape = jax.ShapeDtypeStruct((2, N, H // 2), jnp.float32)
    else:
        out_spec = pl.BlockSpec((BN, H), lambda i: (i, 0))
        out_shape = jax.ShapeDtypeStruct((N, H), jnp.float32)

    return pl.pallas_call(
        body,
        grid=(N // BN,),
        in_specs=in_specs,
        out_specs=out_spec,
        out_shape=out_shape,
    )


_layer1 = _make_layer(D, True)
_layer2 = _make_layer(H, False)


def _combine_body(p_ref, wc_ref, bc_ref, out_ref):
    hg = jnp.max(p_ref[...], axis=0)
    out_ref[...] = (
        jnp.dot(hg, wc_ref[...], preferred_element_type=jnp.float32)
        + bc_ref[...])


_combine = pl.pallas_call(
    _combine_body,
    out_shape=jax.ShapeDtypeStruct((G, C), jnp.float32),
)


def kernel(h, edge_index, graph_ids, W1, b1, W2, b2, Wc, bc):
    src = edge_index[0]
    dst = edge_index[1]

    degp = _degrees(src, dst)
    norms = _norms(degp)
    nt = jnp.transpose(norms)
    nsrc = nt[:, 0:1]
    ndst = nt[:, 1:2]

    xs1 = _prep(h, nsrc)
    agg1 = _spmm_edges(xs1, src, dst)[:, :N]
    xs2 = _layer1(agg1, ndst, W1, b1.reshape(1, H), nsrc)
    agg2 = _spmm_feats(xs2.reshape(2 * N, H // 2), src, dst)[:, :N]
    x2 = _layer2(agg2, ndst, W2, b2.reshape(1, H))

    gpad = jnp.concatenate(
        [graph_ids.astype(jnp.int32),
         jnp.zeros((NPAD - N,), jnp.int32)])
    parts = _segmax(x2, gpad)
    return _combine(parts.reshape(NW, G, H), Wc, bc.reshape(1, C))


# speedup vs baseline: 9.7473x; 2.1971x over previous
import functools

import jax
import jax.numpy as jnp
from jax import lax
from jax.experimental import pallas as pl
from jax.experimental.pallas import tpu as pltpu
from jax.experimental.pallas import tpu_sc as plsc

N = 10000
E = 320000
D = 128
H = 256
C = 32
G = 64

NC, NS, LANES = 2, 16, 16
NW = NC * NS
NPAD = 10240
RPT = NPAD // NS
EK = 64
E2 = 327680
PADE = E2 - E

_mesh = plsc.VectorSubcoreMesh(
    core_axis_name="c", subcore_axis_name="s", num_cores=NC, num_subcores=NS)


DK = 2000
D_EPT = E // NS


@functools.partial(
    pl.kernel,
    out_type=jax.ShapeDtypeStruct((NC, NS, N), jnp.int32),
    mesh=_mesh,
    scratch_types=[
        pltpu.VMEM((DK,), jnp.int32),
        pltpu.VMEM((N,), jnp.int32),
    ],
    compiler_params=pltpu.CompilerParams(needs_layout_passes=False),
)
def _degrees(src_hbm, dst_hbm, out_hbm, idx_v, hist_v):
    c = lax.axis_index("c")
    s = lax.axis_index("s")
    zv = jnp.zeros((LANES,), jnp.int32)

    @pl.loop(0, N // LANES)
    def _(i):
        hist_v[pl.ds(i * LANES, LANES)] = zv

    @pl.loop(0, D_EPT // DK)
    def _(i):
        base = s * D_EPT + i * DK

        @pl.when(c == 0)
        def _():
            pltpu.sync_copy(src_hbm.at[pl.ds(base, DK)], idx_v)

        @pl.when(c == 1)
        def _():
            pltpu.sync_copy(dst_hbm.at[pl.ds(base, DK)], idx_v)

        ones = jnp.ones((LANES,), jnp.int32)

        @pl.loop(0, DK // LANES)
        def _(j):
            idx = idx_v[pl.ds(j * LANES, LANES)]
            plsc.addupdate_scatter(hist_v, [idx], ones)

    pltpu.sync_copy(hist_v, out_hbm.at[c, s])


def _norm_body(degp_ref, norm_ref):
    d = jnp.sum(degp_ref[...].astype(jnp.float32), axis=1)
    norm_ref[...] = jnp.where(d > 0, lax.rsqrt(jnp.maximum(d, 1.0)), 0.0)


_norms = pl.pallas_call(
    _norm_body,
    out_shape=jax.ShapeDtypeStruct((NC, N), jnp.float32),
)


SLOTS = 4
LOOK = 2
STG = 16


def _make_spmm(feat_split):
    ept = E2 // NS if feat_split else E2 // NW
    nchunks = ept // EK
    nstages = nchunks // STG
    assert nstages * STG == nchunks

    @functools.partial(
        pl.kernel,
        out_type=jax.ShapeDtypeStruct((NC, NPAD, 128), jnp.float32),
        mesh=_mesh,
        scratch_types=[
            pltpu.VMEM((2, STG, EK), jnp.int32),
            pltpu.VMEM((2, STG, EK), jnp.int32),
            pltpu.VMEM((SLOTS, EK, 128), jnp.float32),
            pltpu.VMEM_SHARED((NPAD, 128), jnp.float32),
        ] + [pltpu.SemaphoreType.DMA] * (2 * SLOTS),
    )
    def spmm(xs_hbm, src2_hbm, dst2_hbm, out_hbm,
             sstg_v, dstg_v, rows_v, acc_sh, *sems):
        gsem = sems[:SLOTS]
        ssem = sems[SLOTS:]
        c = lax.axis_index("c")
        s = lax.axis_index("s")
        zv = jnp.zeros((LANES,), jnp.float32)

        @pl.loop(0, EK)
        def _(j):
            for k in range(128 // LANES):
                rows_v[0, j, pl.ds(k * LANES, LANES)] = zv

        @pl.loop(0, RPT // EK)
        def _(t):
            pltpu.sync_copy(rows_v.at[0],
                            acc_sh.at[pl.ds(s * RPT + t * EK, EK)])

        plsc.subcore_barrier()
        if feat_split:
            rbase = s * nchunks
            splane = c
        else:
            rbase = (s * NC + c) * nchunks
            splane = 0

        def load_stage(stage):
            par = stage % 2
            row0 = rbase + stage * STG
            pltpu.sync_copy(src2_hbm.at[splane, pl.ds(row0, STG)],
                            sstg_v.at[par])
            pltpu.sync_copy(dst2_hbm.at[pl.ds(row0, STG)], dstg_v.at[par])

        gdesc = {}
        sdesc = {}
        for j in range(nchunks + LOOK):
            if j < nchunks:
                stage, r = divmod(j, STG)
                if r == 0:
                    load_stage(stage)
                slot = j % SLOTS
                if j >= SLOTS:
                    sdesc.pop(j - SLOTS).wait()
                gdesc[j] = pltpu.async_copy(
                    xs_hbm.at[sstg_v.at[stage % 2, r]],
                    rows_v.at[slot], gsem[slot])
            i = j - LOOK
            if i >= 0:
                stage_i, r_i = divmod(i, STG)
                gdesc.pop(i).wait()
                sdesc[i] = pltpu.async_copy(
                    rows_v.at[i % SLOTS],
                    acc_sh.at[dstg_v.at[stage_i % 2, r_i]],
                    ssem[i % SLOTS], add=True)

        for i in sorted(sdesc):
            sdesc[i].wait()

        plsc.subcore_barrier()
        pltpu.sync_copy(acc_sh.at[pl.ds(s * RPT, RPT)],
                        out_hbm.at[c, pl.ds(s * RPT, RPT)])

    return spmm


_spmm_edges = _make_spmm(False)
_spmm_feats = _make_spmm(True)


GB = 16
NGRP = N // GB
GPW = 20
WROWS = GPW * GB
GH = G * H


@functools.partial(
    pl.kernel,
    out_type=jax.ShapeDtypeStruct((NW, GH), jnp.float32),
    mesh=_mesh,
    scratch_types=[
        pltpu.VMEM((WROWS,), jnp.int32),
        pltpu.VMEM((GB, H), jnp.float32),
        pltpu.VMEM((GH,), jnp.float32),
    ],
)
def _segmax(x_hbm, gid_hbm, out_hbm, gid_v, rows_v, acc_v):
    c = lax.axis_index("c")
    s = lax.axis_index("s")
    w = s * NC + c
    ninf = jnp.full((LANES,), -jnp.inf, jnp.float32)

    @pl.loop(0, GH // LANES)
    def _(i):
        acc_v[pl.ds(i * LANES, LANES)] = ninf

    pltpu.sync_copy(gid_hbm.at[pl.ds(w * WROWS, WROWS)], gid_v)
    ngroups = jnp.minimum(GPW, NGRP - w * GPW)

    def group_body(g, carry):
        prev_gid = carry[0]
        regs = list(carry[1:])
        r0 = w * WROWS + g * GB
        pltpu.sync_copy(x_hbm.at[pl.ds(r0, GB)], rows_v)
        gv = gid_v[pl.ds(g * GB, GB)]
        for jj in range(GB):
            gid_j = gv[jj]
            changed = gid_j != prev_gid

            @pl.when(changed & (prev_gid >= 0))
            def _(prev_gid=prev_gid, regs=tuple(regs)):
                for ch in range(H // LANES):
                    acc_v[pl.ds(prev_gid * H + ch * LANES, LANES)] = regs[ch]

            regs = [
                jnp.maximum(jnp.where(changed, ninf, r),
                            rows_v[jj, pl.ds(ch * LANES, LANES)])
                for ch, r in enumerate(regs)
            ]
            prev_gid = gid_j
        return (prev_gid, *regs)

    final = lax.fori_loop(0, ngroups, group_body,
                          (jnp.int32(-1), *([ninf] * (H // LANES))))
    prev_gid = final[0]
    regs = final[1:]

    @pl.when(prev_gid >= 0)
    def _():
        for ch in range(H // LANES):
            acc_v[pl.ds(prev_gid * H + ch * LANES, LANES)] = regs[ch]

    pltpu.sync_copy(acc_v, out_hbm.at[w])


BN = 2000


def _prep_body(h_ref, nsrc_ref, xs_ref):
    xs_ref[...] = h_ref[...] * nsrc_ref[...]


_prep = pl.pallas_call(
    _prep_body,
    grid=(N // BN,),
    in_specs=[
        pl.BlockSpec((BN, D), lambda i: (i, 0)),
        pl.BlockSpec((BN, 1), lambda i: (i, 0)),
    ],
    out_specs=pl.BlockSpec((BN, D), lambda i: (i, 0)),
    out_shape=jax.ShapeDtypeStruct((N, D), jnp.float32),
)


def _make_layer(din, scale_out):
    sum_parts = din == 128

    def body(*refs):
        if scale_out:
            agg_ref, ndst_ref, w_ref, b_ref, nsrc_ref, out_ref = refs
        else:
            agg_ref, ndst_ref, w_ref, b_ref, out_ref = refs
        if sum_parts:
            x = agg_ref[0] + agg_ref[1]
        else:
            x = jnp.concatenate([agg_ref[0], agg_ref[1]], axis=1)
        x = x * ndst_ref[...]
        y = jnp.dot(x, w_ref[...], preferred_element_type=jnp.float32)
        y = jnp.maximum(y + b_ref[...], 0.0)
        if scale_out:
            y = y * nsrc_ref[...]
            out_ref[0] = y[:, :H // 2]
            out_ref[1] = y[:, H // 2:]
        else:
            out_ref[...] = y

    in_specs = [
        pl.BlockSpec((2, BN, 128), lambda i: (0, i, 0)),
        pl.BlockSpec((BN, 1), lambda i: (i, 0)),
        pl.BlockSpec((din, H), lambda i: (0, 0)),
        pl.BlockSpec((1, H), lambda i: (0, 0)),
    ]
    if scale_out:
        in_specs.append(pl.BlockSpec((BN, 1), lambda i: (i, 0)))
        out_spec = pl.BlockSpec((2, BN, H // 2), lambda i: (0, i, 0))
        out_shape = jax.ShapeDtypeStruct((2, N, H // 2), jnp.float32)
    else:
        out_spec = pl.BlockSpec((BN, H), lambda i: (i, 0))
        out_shape = jax.ShapeDtypeStruct((N, H), jnp.float32)

    return pl.pallas_call(
        body,
        grid=(N // BN,),
        in_specs=in_specs,
        out_specs=out_spec,
        out_shape=out_shape,
    )


_layer1 = _make_layer(D, True)
_layer2 = _make_layer(H, False)


def _combine_body(p_ref, wc_ref, bc_ref, out_ref):
    hg = jnp.max(p_ref[...], axis=0)
    out_ref[...] = (
        jnp.dot(hg, wc_ref[...], preferred_element_type=jnp.float32)
        + bc_ref[...])


_combine = pl.pallas_call(
    _combine_body,
    out_shape=jax.ShapeDtypeStruct((G, C), jnp.float32),
)


def kernel(h, edge_index, graph_ids, W1, b1, W2, b2, Wc, bc):
    src = edge_index[0]
    dst = edge_index[1]

    degp = _degrees(src, dst)
    norms = _norms(degp)
    nt = jnp.transpose(norms)
    nsrc = nt[:, 0:1]
    ndst = nt[:, 1:2]

    epad = jnp.arange(PADE, dtype=jnp.int32)
    src_p = jnp.concatenate([src, epad % N])
    dst_p = jnp.concatenate([dst, epad % (NPAD - N) + N])
    src2 = jnp.stack([src_p, src_p + N]).reshape(2, E2 // EK, EK)
    dst2 = dst_p.reshape(E2 // EK, EK)
    xs1 = _prep(h, nsrc)
    agg1 = _spmm_edges(xs1, src2, dst2)[:, :N]
    xs2 = _layer1(agg1, ndst, W1, b1.reshape(1, H), nsrc)
    agg2 = _spmm_feats(xs2.reshape(2 * N, H // 2), src2, dst2)[:, :N]
    x2 = _layer2(agg2, ndst, W2, b2.reshape(1, H))

    gpad = jnp.concatenate(
        [graph_ids.astype(jnp.int32),
         jnp.zeros((NPAD - N,), jnp.int32)])
    parts = _segmax(x2, gpad)
    return _combine(parts.reshape(NW, G, H), Wc, bc.reshape(1, C))


# speedup vs baseline: 10.4689x; 1.0740x over previous
import functools

import jax
import jax.numpy as jnp
from jax import lax
from jax.experimental import pallas as pl
from jax.experimental.pallas import tpu as pltpu
from jax.experimental.pallas import tpu_sc as plsc

N = 10000
E = 320000
D = 128
H = 256
C = 32
G = 64

NC, NS, LANES = 2, 16, 16
NW = NC * NS
NPAD = 10240
RPT = NPAD // NS
EK = 64
E2 = 327680
PADE = E2 - E

_mesh = plsc.VectorSubcoreMesh(
    core_axis_name="c", subcore_axis_name="s", num_cores=NC, num_subcores=NS)


DK = 2000
D_EPT = E // NS


@functools.partial(
    pl.kernel,
    out_type=jax.ShapeDtypeStruct((NC, NS, N), jnp.int32),
    mesh=_mesh,
    scratch_types=[
        pltpu.VMEM((DK,), jnp.int32),
        pltpu.VMEM((N,), jnp.int32),
    ],
    compiler_params=pltpu.CompilerParams(needs_layout_passes=False),
)
def _degrees(src_hbm, dst_hbm, out_hbm, idx_v, hist_v):
    c = lax.axis_index("c")
    s = lax.axis_index("s")
    zv = jnp.zeros((LANES,), jnp.int32)

    @pl.loop(0, N // LANES)
    def _(i):
        hist_v[pl.ds(i * LANES, LANES)] = zv

    @pl.loop(0, D_EPT // DK)
    def _(i):
        base = s * D_EPT + i * DK

        @pl.when(c == 0)
        def _():
            pltpu.sync_copy(src_hbm.at[pl.ds(base, DK)], idx_v)

        @pl.when(c == 1)
        def _():
            pltpu.sync_copy(dst_hbm.at[pl.ds(base, DK)], idx_v)

        ones = jnp.ones((LANES,), jnp.int32)

        @pl.loop(0, DK // LANES)
        def _(j):
            idx = idx_v[pl.ds(j * LANES, LANES)]
            plsc.addupdate_scatter(hist_v, [idx], ones)

    pltpu.sync_copy(hist_v, out_hbm.at[c, s])


def _norm_body(degp_ref, norm_ref):
    d = jnp.sum(degp_ref[...].astype(jnp.float32), axis=1)
    norm_ref[...] = jnp.where(d > 0, lax.rsqrt(jnp.maximum(d, 1.0)), 0.0)


_norms = pl.pallas_call(
    _norm_body,
    out_shape=jax.ShapeDtypeStruct((NC, N), jnp.float32),
)


SLOTS = 5
LOOK = 3
STG = 16


def _make_spmm(feat_split):
    ept = E2 // NS if feat_split else E2 // NW
    nchunks = ept // EK
    nstages = nchunks // STG
    assert nstages * STG == nchunks

    @functools.partial(
        pl.kernel,
        out_type=jax.ShapeDtypeStruct((NC, NPAD, 128), jnp.float32),
        mesh=_mesh,
        scratch_types=[
            pltpu.VMEM((2, STG, EK), jnp.int32),
            pltpu.VMEM((2, STG, EK), jnp.int32),
            pltpu.VMEM((SLOTS, EK, 128), jnp.float32),
            pltpu.VMEM_SHARED((NPAD, 128), jnp.float32),
        ] + [pltpu.SemaphoreType.DMA] * (2 * SLOTS),
    )
    def spmm(xs_hbm, src2_hbm, dst2_hbm, out_hbm,
             sstg_v, dstg_v, rows_v, acc_sh, *sems):
        gsem = sems[:SLOTS]
        ssem = sems[SLOTS:]
        c = lax.axis_index("c")
        s = lax.axis_index("s")
        zv = jnp.zeros((LANES,), jnp.float32)

        @pl.loop(0, EK)
        def _(j):
            for k in range(128 // LANES):
                rows_v[0, j, pl.ds(k * LANES, LANES)] = zv

        @pl.loop(0, RPT // EK)
        def _(t):
            pltpu.sync_copy(rows_v.at[0],
                            acc_sh.at[pl.ds(s * RPT + t * EK, EK)])

        plsc.subcore_barrier()
        if feat_split:
            rbase = s * nchunks
            splane = c
        else:
            rbase = (s * NC + c) * nchunks
            splane = 0

        def load_stage(stage):
            par = stage % 2
            row0 = rbase + stage * STG
            pltpu.sync_copy(src2_hbm.at[splane, pl.ds(row0, STG)],
                            sstg_v.at[par])
            pltpu.sync_copy(dst2_hbm.at[pl.ds(row0, STG)], dstg_v.at[par])

        gdesc = {}
        sdesc = {}
        for j in range(nchunks + LOOK):
            if j < nchunks:
                stage, r = divmod(j, STG)
                if r == 0:
                    load_stage(stage)
                slot = j % SLOTS
                if j >= SLOTS:
                    sdesc.pop(j - SLOTS).wait()
                gdesc[j] = pltpu.async_copy(
                    xs_hbm.at[sstg_v.at[stage % 2, r]],
                    rows_v.at[slot], gsem[slot])
            i = j - LOOK
            if i >= 0:
                stage_i, r_i = divmod(i, STG)
                gdesc.pop(i).wait()
                sdesc[i] = pltpu.async_copy(
                    rows_v.at[i % SLOTS],
                    acc_sh.at[dstg_v.at[stage_i % 2, r_i]],
                    ssem[i % SLOTS], add=True)

        for i in sorted(sdesc):
            sdesc[i].wait()

        plsc.subcore_barrier()
        pltpu.sync_copy(acc_sh.at[pl.ds(s * RPT, RPT)],
                        out_hbm.at[c, pl.ds(s * RPT, RPT)])

    return spmm


_spmm_edges = _make_spmm(False)
_spmm_feats = _make_spmm(True)


GB = 16
NGRP = N // GB
GPW = 20
WROWS = GPW * GB
GH = G * H


@functools.partial(
    pl.kernel,
    out_type=jax.ShapeDtypeStruct((NW, GH), jnp.float32),
    mesh=_mesh,
    scratch_types=[
        pltpu.VMEM((WROWS,), jnp.int32),
        pltpu.VMEM((GB, H), jnp.float32),
        pltpu.VMEM((GH,), jnp.float32),
    ],
)
def _segmax(x_hbm, gid_hbm, out_hbm, gid_v, rows_v, acc_v):
    c = lax.axis_index("c")
    s = lax.axis_index("s")
    w = s * NC + c
    ninf = jnp.full((LANES,), -jnp.inf, jnp.float32)

    @pl.loop(0, GH // LANES)
    def _(i):
        acc_v[pl.ds(i * LANES, LANES)] = ninf

    pltpu.sync_copy(gid_hbm.at[pl.ds(w * WROWS, WROWS)], gid_v)
    ngroups = jnp.minimum(GPW, NGRP - w * GPW)

    def group_body(g, carry):
        prev_gid = carry[0]
        regs = list(carry[1:])
        r0 = w * WROWS + g * GB
        pltpu.sync_copy(x_hbm.at[pl.ds(r0, GB)], rows_v)
        gv = gid_v[pl.ds(g * GB, GB)]
        for jj in range(GB):
            gid_j = gv[jj]
            changed = gid_j != prev_gid

            @pl.when(changed & (prev_gid >= 0))
            def _(prev_gid=prev_gid, regs=tuple(regs)):
                for ch in range(H // LANES):
                    acc_v[pl.ds(prev_gid * H + ch * LANES, LANES)] = regs[ch]

            regs = [
                jnp.maximum(jnp.where(changed, ninf, r),
                            rows_v[jj, pl.ds(ch * LANES, LANES)])
                for ch, r in enumerate(regs)
            ]
            prev_gid = gid_j
        return (prev_gid, *regs)

    final = lax.fori_loop(0, ngroups, group_body,
                          (jnp.int32(-1), *([ninf] * (H // LANES))))
    prev_gid = final[0]
    regs = final[1:]

    @pl.when(prev_gid >= 0)
    def _():
        for ch in range(H // LANES):
            acc_v[pl.ds(prev_gid * H + ch * LANES, LANES)] = regs[ch]

    pltpu.sync_copy(acc_v, out_hbm.at[w])


BN = 2000


def _prep_body(h_ref, nsrc_ref, xs_ref):
    xs_ref[...] = h_ref[...] * nsrc_ref[...]


_prep = pl.pallas_call(
    _prep_body,
    grid=(N // BN,),
    in_specs=[
        pl.BlockSpec((BN, D), lambda i: (i, 0)),
        pl.BlockSpec((BN, 1), lambda i: (i, 0)),
    ],
    out_specs=pl.BlockSpec((BN, D), lambda i: (i, 0)),
    out_shape=jax.ShapeDtypeStruct((N, D), jnp.float32),
)


def _make_layer(din, scale_out):
    sum_parts = din == 128

    def body(*refs):
        if scale_out:
            agg_ref, ndst_ref, w_ref, b_ref, nsrc_ref, out_ref = refs
        else:
            agg_ref, ndst_ref, w_ref, b_ref, out_ref = refs
        if sum_parts:
            x = agg_ref[0] + agg_ref[1]
        else:
            x = jnp.concatenate([agg_ref[0], agg_ref[1]], axis=1)
        x = x * ndst_ref[...]
        y = jnp.dot(x, w_ref[...], preferred_element_type=jnp.float32)
        y = jnp.maximum(y + b_ref[...], 0.0)
        if scale_out:
            y = y * nsrc_ref[...]
            out_ref[0] = y[:, :H // 2]
            out_ref[1] = y[:, H // 2:]
        else:
            out_ref[...] = y

    in_specs = [
        pl.BlockSpec((2, BN, 128), lambda i: (0, i, 0)),
        pl.BlockSpec((BN, 1), lambda i: (i, 0)),
        pl.BlockSpec((din, H), lambda i: (0, 0)),
        pl.BlockSpec((1, H), lambda i: (0, 0)),
    ]
    if scale_out:
        in_specs.append(pl.BlockSpec((BN, 1), lambda i: (i, 0)))
        out_spec = pl.BlockSpec((2, BN, H // 2), lambda i: (0, i, 0))
        out_shape = jax.ShapeDtypeStruct((2, N, H // 2), jnp.float32)
    else:
        out_spec = pl.BlockSpec((BN, H), lambda i: (i, 0))
        out_shape = jax.ShapeDtypeStruct((N, H), jnp.float32)

    return pl.pallas_call(
        body,
        grid=(N // BN,),
        in_specs=in_specs,
        out_specs=out_spec,
        out_shape=out_shape,
    )


_layer1 = _make_layer(D, True)
_layer2 = _make_layer(H, False)


def _combine_body(p_ref, wc_ref, bc_ref, out_ref):
    hg = jnp.max(p_ref[...], axis=0)
    out_ref[...] = (
        jnp.dot(hg, wc_ref[...], preferred_element_type=jnp.float32)
        + bc_ref[...])


_combine = pl.pallas_call(
    _combine_body,
    out_shape=jax.ShapeDtypeStruct((G, C), jnp.float32),
)


def kernel(h, edge_index, graph_ids, W1, b1, W2, b2, Wc, bc):
    src = edge_index[0]
    dst = edge_index[1]

    degp = _degrees(src, dst)
    norms = _norms(degp)
    nt = jnp.transpose(norms)
    nsrc = nt[:, 0:1]
    ndst = nt[:, 1:2]

    epad = jnp.arange(PADE, dtype=jnp.int32)
    src_p = jnp.concatenate([src, epad % N])
    dst_p = jnp.concatenate([dst, epad % (NPAD - N) + N])
    src2 = jnp.stack([src_p, src_p + N]).reshape(2, E2 // EK, EK)
    dst2 = dst_p.reshape(E2 // EK, EK)
    xs1 = _prep(h, nsrc)
    agg1 = _spmm_edges(xs1, src2, dst2)
    xs2 = _layer1(agg1, ndst, W1, b1.reshape(1, H), nsrc)
    agg2 = _spmm_feats(xs2.reshape(2 * N, H // 2), src2, dst2)
    x2 = _layer2(agg2, ndst, W2, b2.reshape(1, H))

    gpad = jnp.concatenate(
        [graph_ids.astype(jnp.int32),
         jnp.zeros((NPAD - N,), jnp.int32)])
    parts = _segmax(x2, gpad)
    return _combine(parts.reshape(NW, G, H), Wc, bc.reshape(1, C))


# speedup vs baseline: 10.8198x; 1.0335x over previous
import functools

import jax
import jax.numpy as jnp
from jax import lax
from jax.experimental import pallas as pl
from jax.experimental.pallas import tpu as pltpu
from jax.experimental.pallas import tpu_sc as plsc

N = 10000
E = 320000
D = 128
H = 256
C = 32
G = 64

NC, NS, LANES = 2, 16, 16
NW = NC * NS
NPAD = 10240
RPT = NPAD // NS
EK = 64
E2 = 327680
PADE = E2 - E

_mesh = plsc.VectorSubcoreMesh(
    core_axis_name="c", subcore_axis_name="s", num_cores=NC, num_subcores=NS)


DK = 2000
D_EPT = E // NS


DCH = D_EPT // DK


@functools.partial(
    pl.kernel,
    out_type=jax.ShapeDtypeStruct((NC, NS, N), jnp.int32),
    mesh=_mesh,
    scratch_types=[
        pltpu.VMEM((DK,), jnp.int32),
        pltpu.VMEM((DK,), jnp.int32),
        pltpu.VMEM((N,), jnp.int32),
        pltpu.SemaphoreType.DMA,
        pltpu.SemaphoreType.DMA,
    ],
    compiler_params=pltpu.CompilerParams(needs_layout_passes=False),
)
def _degrees(src_hbm, dst_hbm, out_hbm, idx_a, idx_b, hist_v, sem0, sem1):
    c = lax.axis_index("c")
    s = lax.axis_index("s")
    zv = jnp.zeros((LANES,), jnp.int32)
    ones = jnp.ones((LANES,), jnp.int32)
    sems = (sem0, sem1)
    bufs = (idx_a, idx_b)

    @pl.loop(0, N // LANES)
    def _(i):
        hist_v[pl.ds(i * LANES, LANES)] = zv

    def load(i):
        base = s * D_EPT + i * DK
        slot = i % 2

        @pl.when(c == 0)
        def _():
            pltpu.async_copy(src_hbm.at[pl.ds(base, DK)],
                             bufs[slot], sems[slot])

        @pl.when(c == 1)
        def _():
            pltpu.async_copy(dst_hbm.at[pl.ds(base, DK)],
                             bufs[slot], sems[slot])

        return pltpu.make_async_copy(src_hbm.at[pl.ds(base, DK)],
                                     bufs[slot], sems[slot])

    pend = {0: load(0)}
    for i in range(DCH):
        if i + 1 < DCH:
            pend[i + 1] = load(i + 1)
        pend.pop(i).wait()
        par = i % 2

        buf = bufs[par]

        @pl.loop(0, DK // LANES)
        def _(j, buf=buf):
            idx = buf[pl.ds(j * LANES, LANES)]
            plsc.addupdate_scatter(hist_v, [idx], ones)

    pltpu.sync_copy(hist_v, out_hbm.at[c, s])


SLOTS = 5
LOOK = 3
STG = 16


def _make_spmm(feat_split):
    ept = E2 // NS if feat_split else E2 // NW
    nchunks = ept // EK
    nstages = nchunks // STG
    assert nstages * STG == nchunks

    @functools.partial(
        pl.kernel,
        out_type=jax.ShapeDtypeStruct((NC, NPAD, 128), jnp.float32),
        mesh=_mesh,
        scratch_types=[
            pltpu.VMEM((2, STG, EK), jnp.int32),
            pltpu.VMEM((2, STG, EK), jnp.int32),
            pltpu.VMEM((SLOTS, EK, 128), jnp.float32),
            pltpu.VMEM_SHARED((NPAD, 128), jnp.float32),
        ] + [pltpu.SemaphoreType.DMA] * (2 * SLOTS),
    )
    def spmm(xs_hbm, src2_hbm, dst2_hbm, out_hbm,
             sstg_v, dstg_v, rows_v, acc_sh, *sems):
        gsem = sems[:SLOTS]
        ssem = sems[SLOTS:]
        c = lax.axis_index("c")
        s = lax.axis_index("s")
        zv = jnp.zeros((LANES,), jnp.float32)

        @pl.loop(0, EK)
        def _(j):
            for k in range(128 // LANES):
                rows_v[0, j, pl.ds(k * LANES, LANES)] = zv

        @pl.loop(0, RPT // EK)
        def _(t):
            pltpu.sync_copy(rows_v.at[0],
                            acc_sh.at[pl.ds(s * RPT + t * EK, EK)])

        plsc.subcore_barrier()
        if feat_split:
            rbase = s * nchunks
            splane = c
        else:
            rbase = (s * NC + c) * nchunks
            splane = 0

        def load_stage(stage):
            par = stage % 2
            row0 = rbase + stage * STG
            pltpu.sync_copy(src2_hbm.at[splane, pl.ds(row0, STG)],
                            sstg_v.at[par])
            pltpu.sync_copy(dst2_hbm.at[pl.ds(row0, STG)], dstg_v.at[par])

        gdesc = {}
        sdesc = {}
        for j in range(nchunks + LOOK):
            if j < nchunks:
                stage, r = divmod(j, STG)
                if r == 0:
                    load_stage(stage)
                slot = j % SLOTS
                if j >= SLOTS:
                    sdesc.pop(j - SLOTS).wait()
                gdesc[j] = pltpu.async_copy(
                    xs_hbm.at[sstg_v.at[stage % 2, r]],
                    rows_v.at[slot], gsem[slot])
            i = j - LOOK
            if i >= 0:
                stage_i, r_i = divmod(i, STG)
                gdesc.pop(i).wait()
                sdesc[i] = pltpu.async_copy(
                    rows_v.at[i % SLOTS],
                    acc_sh.at[dstg_v.at[stage_i % 2, r_i]],
                    ssem[i % SLOTS], add=True)

        for i in sorted(sdesc):
            sdesc[i].wait()

        plsc.subcore_barrier()
        pltpu.sync_copy(acc_sh.at[pl.ds(s * RPT, RPT)],
                        out_hbm.at[c, pl.ds(s * RPT, RPT)])

    return spmm


_spmm_edges = _make_spmm(False)
_spmm_feats = _make_spmm(True)


GB = 16
NGRP = N // GB
GPW = 20
WROWS = GPW * GB
GH = G * H


@functools.partial(
    pl.kernel,
    out_type=jax.ShapeDtypeStruct((NW, GH), jnp.float32),
    mesh=_mesh,
    scratch_types=[
        pltpu.VMEM((WROWS,), jnp.int32),
        pltpu.VMEM((GB, H), jnp.float32),
        pltpu.VMEM((GH,), jnp.float32),
    ],
)
def _segmax(x_hbm, gid_hbm, out_hbm, gid_v, rows_v, acc_v):
    c = lax.axis_index("c")
    s = lax.axis_index("s")
    w = s * NC + c
    ninf = jnp.full((LANES,), -jnp.inf, jnp.float32)

    @pl.loop(0, GH // LANES)
    def _(i):
        acc_v[pl.ds(i * LANES, LANES)] = ninf

    pltpu.sync_copy(gid_hbm.at[pl.ds(w * WROWS, WROWS)], gid_v)
    ngroups = jnp.minimum(GPW, NGRP - w * GPW)

    def group_body(g, carry):
        prev_gid = carry[0]
        regs = list(carry[1:])
        r0 = w * WROWS + g * GB
        pltpu.sync_copy(x_hbm.at[pl.ds(r0, GB)], rows_v)
        gv = gid_v[pl.ds(g * GB, GB)]
        for jj in range(GB):
            gid_j = gv[jj]
            changed = gid_j != prev_gid

            @pl.when(changed & (prev_gid >= 0))
            def _(prev_gid=prev_gid, regs=tuple(regs)):
                for ch in range(H // LANES):
                    acc_v[pl.ds(prev_gid * H + ch * LANES, LANES)] = regs[ch]

            regs = [
                jnp.maximum(jnp.where(changed, ninf, r),
                            rows_v[jj, pl.ds(ch * LANES, LANES)])
                for ch, r in enumerate(regs)
            ]
            prev_gid = gid_j
        return (prev_gid, *regs)

    final = lax.fori_loop(0, ngroups, group_body,
                          (jnp.int32(-1), *([ninf] * (H // LANES))))
    prev_gid = final[0]
    regs = final[1:]

    @pl.when(prev_gid >= 0)
    def _():
        for ch in range(H // LANES):
            acc_v[pl.ds(prev_gid * H + ch * LANES, LANES)] = regs[ch]

    pltpu.sync_copy(acc_v, out_hbm.at[w])


BN = 2000


def _prep_body(degp_ref, h_ref, nt_ref, xs_ref):
    d = jnp.sum(degp_ref[...].astype(jnp.float32), axis=1)
    nm = jnp.where(d > 0, lax.rsqrt(jnp.maximum(d, 1.0)), 0.0)
    nt = nm.T
    nt_ref[...] = nt
    xs_ref[...] = h_ref[...] * nt[:, 0:1]


_prep = pl.pallas_call(
    _prep_body,
    out_shape=[
        jax.ShapeDtypeStruct((N, 2), jnp.float32),
        jax.ShapeDtypeStruct((N, D), jnp.float32),
    ],
)


def _make_layer(din, scale_out):
    sum_parts = din == 128

    def body(*refs):
        if scale_out:
            agg_ref, ndst_ref, w_ref, b_ref, nsrc_ref, out_ref = refs
        else:
            agg_ref, ndst_ref, w_ref, b_ref, out_ref = refs
        if sum_parts:
            x = agg_ref[0] + agg_ref[1]
        else:
            x = jnp.concatenate([agg_ref[0], agg_ref[1]], axis=1)
        x = x * ndst_ref[...]
        y = jnp.dot(x, w_ref[...], preferred_element_type=jnp.float32)
        y = jnp.maximum(y + b_ref[...], 0.0)
        if scale_out:
            y = y * nsrc_ref[...]
            out_ref[0] = y[:, :H // 2]
            out_ref[1] = y[:, H // 2:]
        else:
            out_ref[...] = y

    in_specs = [
        pl.BlockSpec((2, BN, 128), lambda i: (0, i, 0)),
        pl.BlockSpec((BN, 1), lambda i: (i, 0)),
        pl.BlockSpec((din, H), lambda i: (0, 0)),
        pl.BlockSpec((1, H), lambda i: (0, 0)),
    ]
    if scale_out:
        in_specs.append(pl.BlockSpec((BN, 1), lambda i: (i, 0)))
        out_spec = pl.BlockSpec((2, BN, H // 2), lambda i: (0, i, 0))
        out_shape = jax.ShapeDtypeStruct((2, N, H // 2), jnp.float32)
    else:
        out_spec = pl.BlockSpec((BN, H), lambda i: (i, 0))
        out_shape = jax.ShapeDtypeStruct((N, H), jnp.float32)

    return pl.pallas_call(
        body,
        grid=(N // BN,),
        in_specs=in_specs,
        out_specs=out_spec,
        out_shape=out_shape,
    )


_layer1 = _make_layer(D, True)
_layer2 = _make_layer(H, False)


def _combine_body(p_ref, wc_ref, bc_ref, out_ref):
    hg = jnp.max(p_ref[...], axis=0)
    out_ref[...] = (
        jnp.dot(hg, wc_ref[...], preferred_element_type=jnp.float32)
        + bc_ref[...])


_combine = pl.pallas_call(
    _combine_body,
    out_shape=jax.ShapeDtypeStruct((G, C), jnp.float32),
)


def kernel(h, edge_index, graph_ids, W1, b1, W2, b2, Wc, bc):
    src = edge_index[0]
    dst = edge_index[1]

    degp = _degrees(src, dst)
    nt, xs1 = _prep(degp, h)
    nsrc = nt[:, 0:1]
    ndst = nt[:, 1:2]

    epad = jnp.arange(PADE, dtype=jnp.int32)
    src_p = jnp.concatenate([src, epad % N])
    dst_p = jnp.concatenate([dst, epad % (NPAD - N) + N])
    src2 = jnp.stack([src_p, src_p + N]).reshape(2, E2 // EK, EK)
    dst2 = dst_p.reshape(E2 // EK, EK)
    agg1 = _spmm_edges(xs1, src2, dst2)
    xs2 = _layer1(agg1, ndst, W1, b1.reshape(1, H), nsrc)
    agg2 = _spmm_feats(xs2.reshape(2 * N, H // 2), src2, dst2)
    x2 = _layer2(agg2, ndst, W2, b2.reshape(1, H))

    gpad = jnp.concatenate(
        [graph_ids.astype(jnp.int32),
         jnp.zeros((NPAD - N,), jnp.int32)])
    parts = _segmax(x2, gpad)
    return _combine(parts.reshape(NW, G, H), Wc, bc.reshape(1, C))


# speedup vs baseline: 11.4889x; 1.0618x over previous
import functools

import jax
import jax.numpy as jnp
from jax import lax
from jax.experimental import pallas as pl
from jax.experimental.pallas import tpu as pltpu
from jax.experimental.pallas import tpu_sc as plsc

N = 10000
E = 320000
D = 128
H = 256
C = 32
G = 64

NC, NS, LANES = 2, 16, 16
NW = NC * NS
NPAD = 10240
RPT = NPAD // NS
EK = 64
E2 = 327680
PADE = E2 - E

_mesh = plsc.VectorSubcoreMesh(
    core_axis_name="c", subcore_axis_name="s", num_cores=NC, num_subcores=NS)


DK = 2000
D_EPT = E // NS


DCH = D_EPT // DK


@functools.partial(
    pl.kernel,
    out_type=jax.ShapeDtypeStruct((NC, NS, N), jnp.int32),
    mesh=_mesh,
    scratch_types=[
        pltpu.VMEM((DK,), jnp.int32),
        pltpu.VMEM((DK,), jnp.int32),
        pltpu.VMEM((N,), jnp.int32),
        pltpu.SemaphoreType.DMA,
        pltpu.SemaphoreType.DMA,
    ],
    compiler_params=pltpu.CompilerParams(needs_layout_passes=False),
)
def _degrees(src_hbm, dst_hbm, out_hbm, idx_a, idx_b, hist_v, sem0, sem1):
    c = lax.axis_index("c")
    s = lax.axis_index("s")
    zv = jnp.zeros((LANES,), jnp.int32)
    ones = jnp.ones((LANES,), jnp.int32)
    sems = (sem0, sem1)
    bufs = (idx_a, idx_b)

    @pl.loop(0, N // LANES)
    def _(i):
        hist_v[pl.ds(i * LANES, LANES)] = zv

    def load(i):
        base = s * D_EPT + i * DK
        slot = i % 2

        @pl.when(c == 0)
        def _():
            pltpu.async_copy(src_hbm.at[pl.ds(base, DK)],
                             bufs[slot], sems[slot])

        @pl.when(c == 1)
        def _():
            pltpu.async_copy(dst_hbm.at[pl.ds(base, DK)],
                             bufs[slot], sems[slot])

        return pltpu.make_async_copy(src_hbm.at[pl.ds(base, DK)],
                                     bufs[slot], sems[slot])

    pend = {0: load(0)}
    for i in range(DCH):
        if i + 1 < DCH:
            pend[i + 1] = load(i + 1)
        pend.pop(i).wait()
        par = i % 2

        buf = bufs[par]

        @pl.loop(0, DK // LANES)
        def _(j, buf=buf):
            idx = buf[pl.ds(j * LANES, LANES)]
            plsc.addupdate_scatter(hist_v, [idx], ones)

    pltpu.sync_copy(hist_v, out_hbm.at[c, s])


SLOTS = 5
LOOK = 3
STG = 16


def _make_spmm(feat_split):
    ept = E2 // NS if feat_split else E2 // NW
    nchunks = ept // EK
    nstages = nchunks // STG
    assert nstages * STG == nchunks

    @functools.partial(
        pl.kernel,
        out_type=jax.ShapeDtypeStruct((NC, NPAD, 128), jnp.float32),
        mesh=_mesh,
        scratch_types=[
            pltpu.VMEM((2, STG, EK), jnp.int32),
            pltpu.VMEM((2, STG, EK), jnp.int32),
            pltpu.VMEM((SLOTS, EK, 128), jnp.float32),
            pltpu.VMEM_SHARED((NPAD, 128), jnp.float32),
        ] + [pltpu.SemaphoreType.DMA] * (2 * SLOTS + 2),
    )
    def spmm(xs_hbm, src2_hbm, dst2_hbm, out_hbm,
             sstg_v, dstg_v, rows_v, acc_sh, *sems):
        gsem = sems[:SLOTS]
        ssem = sems[SLOTS:2 * SLOTS]
        stsem = sems[2 * SLOTS:]
        c = lax.axis_index("c")
        s = lax.axis_index("s")
        zv = jnp.zeros((LANES,), jnp.float32)

        @pl.loop(0, EK)
        def _(j):
            for k in range(128 // LANES):
                rows_v[0, j, pl.ds(k * LANES, LANES)] = zv

        @pl.loop(0, RPT // EK)
        def _(t):
            pltpu.sync_copy(rows_v.at[0],
                            acc_sh.at[pl.ds(s * RPT + t * EK, EK)])

        plsc.subcore_barrier()
        if feat_split:
            rbase = s * nchunks
            splane = c
        else:
            rbase = (s * NC + c) * nchunks
            splane = 0

        def start_stage(stage):
            par = stage % 2
            row0 = rbase + stage * STG
            return (
                pltpu.async_copy(src2_hbm.at[splane, pl.ds(row0, STG)],
                                 sstg_v.at[par], stsem[0]),
                pltpu.async_copy(dst2_hbm.at[pl.ds(row0, STG)],
                                 dstg_v.at[par], stsem[1]),
            )

        stg_pend = {0: start_stage(0)}
        gdesc = {}
        sdesc = {}
        for j in range(nchunks + LOOK):
            if j < nchunks:
                stage, r = divmod(j, STG)
                if r == 0:
                    for d in stg_pend.pop(stage):
                        d.wait()
                if r == SLOTS + LOOK and stage + 1 < nstages:
                    stg_pend[stage + 1] = start_stage(stage + 1)
                slot = j % SLOTS
                if j >= SLOTS:
                    sdesc.pop(j - SLOTS).wait()
                gdesc[j] = pltpu.async_copy(
                    xs_hbm.at[sstg_v.at[stage % 2, r]],
                    rows_v.at[slot], gsem[slot])
            i = j - LOOK
            if i >= 0:
                stage_i, r_i = divmod(i, STG)
                gdesc.pop(i).wait()
                sdesc[i] = pltpu.async_copy(
                    rows_v.at[i % SLOTS],
                    acc_sh.at[dstg_v.at[stage_i % 2, r_i]],
                    ssem[i % SLOTS], add=True)

        for i in sorted(sdesc):
            sdesc[i].wait()

        plsc.subcore_barrier()
        pltpu.sync_copy(acc_sh.at[pl.ds(s * RPT, RPT)],
                        out_hbm.at[c, pl.ds(s * RPT, RPT)])

    return spmm


_spmm_edges = _make_spmm(False)
_spmm_feats = _make_spmm(True)


GB = 16
NGRP = N // GB
GPW = 20
WROWS = GPW * GB
GH = G * H


@functools.partial(
    pl.kernel,
    out_type=jax.ShapeDtypeStruct((NW, GH), jnp.float32),
    mesh=_mesh,
    scratch_types=[
        pltpu.VMEM((WROWS,), jnp.int32),
        pltpu.VMEM((GB, H), jnp.float32),
        pltpu.VMEM((GH,), jnp.float32),
    ],
)
def _segmax(x_hbm, gid_hbm, out_hbm, gid_v, rows_v, acc_v):
    c = lax.axis_index("c")
    s = lax.axis_index("s")
    w = s * NC + c
    ninf = jnp.full((LANES,), -jnp.inf, jnp.float32)

    @pl.loop(0, GH // LANES)
    def _(i):
        acc_v[pl.ds(i * LANES, LANES)] = ninf

    pltpu.sync_copy(gid_hbm.at[pl.ds(w * WROWS, WROWS)], gid_v)
    ngroups = jnp.minimum(GPW, NGRP - w * GPW)

    def group_body(g, carry):
        prev_gid = carry[0]
        regs = list(carry[1:])
        r0 = w * WROWS + g * GB
        pltpu.sync_copy(x_hbm.at[pl.ds(r0, GB)], rows_v)
        gv = gid_v[pl.ds(g * GB, GB)]
        for jj in range(GB):
            gid_j = gv[jj]
            changed = gid_j != prev_gid

            @pl.when(changed & (prev_gid >= 0))
            def _(prev_gid=prev_gid, regs=tuple(regs)):
                for ch in range(H // LANES):
                    acc_v[pl.ds(prev_gid * H + ch * LANES, LANES)] = regs[ch]

            regs = [
                jnp.maximum(jnp.where(changed, ninf, r),
                            rows_v[jj, pl.ds(ch * LANES, LANES)])
                for ch, r in enumerate(regs)
            ]
            prev_gid = gid_j
        return (prev_gid, *regs)

    final = lax.fori_loop(0, ngroups, group_body,
                          (jnp.int32(-1), *([ninf] * (H // LANES))))
    prev_gid = final[0]
    regs = final[1:]

    @pl.when(prev_gid >= 0)
    def _():
        for ch in range(H // LANES):
            acc_v[pl.ds(prev_gid * H + ch * LANES, LANES)] = regs[ch]

    pltpu.sync_copy(acc_v, out_hbm.at[w])


BN = 2000


def _prep_body(degp_ref, h_ref, nt_ref, xs_ref):
    d = jnp.sum(degp_ref[...].astype(jnp.float32), axis=1)
    nm = jnp.where(d > 0, lax.rsqrt(jnp.maximum(d, 1.0)), 0.0)
    nt = nm.T
    nt_ref[...] = nt
    xs_ref[...] = h_ref[...] * nt[:, 0:1]


_prep = pl.pallas_call(
    _prep_body,
    out_shape=[
        jax.ShapeDtypeStruct((N, 2), jnp.float32),
        jax.ShapeDtypeStruct((N, D), jnp.float32),
    ],
)


def _make_layer(din, scale_out):
    sum_parts = din == 128

    def body(*refs):
        if scale_out:
            agg_ref, ndst_ref, w_ref, b_ref, nsrc_ref, out_ref = refs
        else:
            agg_ref, ndst_ref, w_ref, b_ref, out_ref = refs
        if sum_parts:
            x = agg_ref[0] + agg_ref[1]
        else:
            x = jnp.concatenate([agg_ref[0], agg_ref[1]], axis=1)
        x = x * ndst_ref[...]
        y = jnp.dot(x, w_ref[...], preferred_element_type=jnp.float32)
        y = jnp.maximum(y + b_ref[...], 0.0)
        if scale_out:
            y = y * nsrc_ref[...]
            out_ref[0] = y[:, :H // 2]
            out_ref[1] = y[:, H // 2:]
        else:
            out_ref[...] = y

    in_specs = [
        pl.BlockSpec((2, BN, 128), lambda i: (0, i, 0)),
        pl.BlockSpec((BN, 1), lambda i: (i, 0)),
        pl.BlockSpec((din, H), lambda i: (0, 0)),
        pl.BlockSpec((1, H), lambda i: (0, 0)),
    ]
    if scale_out:
        in_specs.append(pl.BlockSpec((BN, 1), lambda i: (i, 0)))
        out_spec = pl.BlockSpec((2, BN, H // 2), lambda i: (0, i, 0))
        out_shape = jax.ShapeDtypeStruct((2, N, H // 2), jnp.float32)
    else:
        out_spec = pl.BlockSpec((BN, H), lambda i: (i, 0))
        out_shape = jax.ShapeDtypeStruct((N, H), jnp.float32)

    return pl.pallas_call(
        body,
        grid=(N // BN,),
        in_specs=in_specs,
        out_specs=out_spec,
        out_shape=out_shape,
    )


_layer1 = _make_layer(D, True)
_layer2 = _make_layer(H, False)


def _combine_body(p_ref, wc_ref, bc_ref, out_ref):
    hg = jnp.max(p_ref[...], axis=0)
    out_ref[...] = (
        jnp.dot(hg, wc_ref[...], preferred_element_type=jnp.float32)
        + bc_ref[...])


_combine = pl.pallas_call(
    _combine_body,
    out_shape=jax.ShapeDtypeStruct((G, C), jnp.float32),
)


def kernel(h, edge_index, graph_ids, W1, b1, W2, b2, Wc, bc):
    src = edge_index[0]
    dst = edge_index[1]

    degp = _degrees(src, dst)
    nt, xs1 = _prep(degp, h)
    nsrc = nt[:, 0:1]
    ndst = nt[:, 1:2]

    epad = jnp.arange(PADE, dtype=jnp.int32)
    src_p = jnp.concatenate([src, epad % N])
    dst_p = jnp.concatenate([dst, epad % (NPAD - N) + N])
    src2 = jnp.stack([src_p, src_p + N]).reshape(2, E2 // EK, EK)
    dst2 = dst_p.reshape(E2 // EK, EK)
    agg1 = _spmm_edges(xs1, src2, dst2)
    xs2 = _layer1(agg1, ndst, W1, b1.reshape(1, H), nsrc)
    agg2 = _spmm_feats(xs2.reshape(2 * N, H // 2), src2, dst2)
    x2 = _layer2(agg2, ndst, W2, b2.reshape(1, H))

    gpad = jnp.concatenate(
        [graph_ids.astype(jnp.int32),
         jnp.zeros((NPAD - N,), jnp.int32)])
    parts = _segmax(x2, gpad)
    return _combine(parts.reshape(NW, G, H), Wc, bc.reshape(1, C))


# speedup vs baseline: 11.6828x; 1.0169x over previous
import functools

import jax
import jax.numpy as jnp
from jax import lax
from jax.experimental import pallas as pl
from jax.experimental.pallas import tpu as pltpu
from jax.experimental.pallas import tpu_sc as plsc

N = 10000
E = 320000
D = 128
H = 256
C = 32
G = 64

NC, NS, LANES = 2, 16, 16
NW = NC * NS
NPAD = 10240
RPT = NPAD // NS
EK = 64
E2 = 327680
PADE = E2 - E

_mesh = plsc.VectorSubcoreMesh(
    core_axis_name="c", subcore_axis_name="s", num_cores=NC, num_subcores=NS)


DK = 2000
D_EPT = E // NS


DCH = D_EPT // DK


@functools.partial(
    pl.kernel,
    out_type=jax.ShapeDtypeStruct((NC, NS, N), jnp.int32),
    mesh=_mesh,
    scratch_types=[
        pltpu.VMEM((DK,), jnp.int32),
        pltpu.VMEM((DK,), jnp.int32),
        pltpu.VMEM((N,), jnp.int32),
        pltpu.SemaphoreType.DMA,
        pltpu.SemaphoreType.DMA,
    ],
    compiler_params=pltpu.CompilerParams(needs_layout_passes=False),
)
def _degrees(src_hbm, dst_hbm, out_hbm, idx_a, idx_b, hist_v, sem0, sem1):
    c = lax.axis_index("c")
    s = lax.axis_index("s")
    zv = jnp.zeros((LANES,), jnp.int32)
    ones = jnp.ones((LANES,), jnp.int32)
    sems = (sem0, sem1)
    bufs = (idx_a, idx_b)

    @pl.loop(0, N // LANES)
    def _(i):
        hist_v[pl.ds(i * LANES, LANES)] = zv

    def load(i):
        base = s * D_EPT + i * DK
        slot = i % 2

        @pl.when(c == 0)
        def _():
            pltpu.async_copy(src_hbm.at[pl.ds(base, DK)],
                             bufs[slot], sems[slot])

        @pl.when(c == 1)
        def _():
            pltpu.async_copy(dst_hbm.at[pl.ds(base, DK)],
                             bufs[slot], sems[slot])

        return pltpu.make_async_copy(src_hbm.at[pl.ds(base, DK)],
                                     bufs[slot], sems[slot])

    pend = {0: load(0)}
    for i in range(DCH):
        if i + 1 < DCH:
            pend[i + 1] = load(i + 1)
        pend.pop(i).wait()
        par = i % 2

        buf = bufs[par]

        @pl.loop(0, DK // LANES)
        def _(j, buf=buf):
            idx = buf[pl.ds(j * LANES, LANES)]
            plsc.addupdate_scatter(hist_v, [idx], ones)

    pltpu.sync_copy(hist_v, out_hbm.at[c, s])


SLOTS = 5
LOOK = 3
STG = 16


def _make_spmm(feat_split):
    ept = E2 // NS if feat_split else E2 // NW
    nchunks = ept // EK
    nstages = nchunks // STG
    assert nstages * STG == nchunks

    @functools.partial(
        pl.kernel,
        out_type=jax.ShapeDtypeStruct((NC, NPAD, 128), jnp.float32),
        mesh=_mesh,
        scratch_types=[
            pltpu.VMEM((2, STG, EK), jnp.int32),
            pltpu.VMEM((2, STG, EK), jnp.int32),
            pltpu.VMEM((SLOTS, EK, 128), jnp.float32),
            pltpu.VMEM_SHARED((NPAD, 128), jnp.float32),
        ] + [pltpu.SemaphoreType.DMA] * (2 * SLOTS + 2),
    )
    def spmm(xs_hbm, src2_hbm, dst2_hbm, out_hbm,
             sstg_v, dstg_v, rows_v, acc_sh, *sems):
        gsem = sems[:SLOTS]
        ssem = sems[SLOTS:2 * SLOTS]
        stsem = sems[2 * SLOTS:]
        c = lax.axis_index("c")
        s = lax.axis_index("s")
        zv = jnp.zeros((LANES,), jnp.float32)

        @pl.loop(0, EK)
        def _(j):
            for k in range(128 // LANES):
                rows_v[0, j, pl.ds(k * LANES, LANES)] = zv

        @pl.loop(0, RPT // EK)
        def _(t):
            pltpu.sync_copy(rows_v.at[0],
                            acc_sh.at[pl.ds(s * RPT + t * EK, EK)])

        plsc.subcore_barrier()
        if feat_split:
            rbase = s * nchunks
            splane = c
        else:
            rbase = (s * NC + c) * nchunks
            splane = 0

        def start_stage(stage):
            par = stage % 2
            row0 = rbase + stage * STG
            return (
                pltpu.async_copy(src2_hbm.at[splane, pl.ds(row0, STG)],
                                 sstg_v.at[par], stsem[0]),
                pltpu.async_copy(dst2_hbm.at[pl.ds(row0, STG)],
                                 dstg_v.at[par], stsem[1]),
            )

        stg_pend = {0: start_stage(0)}
        gdesc = {}
        sdesc = {}
        for j in range(nchunks + LOOK):
            if j < nchunks:
                stage, r = divmod(j, STG)
                if r == 0:
                    for d in stg_pend.pop(stage):
                        d.wait()
                if r == SLOTS + LOOK and stage + 1 < nstages:
                    stg_pend[stage + 1] = start_stage(stage + 1)
                slot = j % SLOTS
                if j >= SLOTS:
                    sdesc.pop(j - SLOTS).wait()
                gdesc[j] = pltpu.async_copy(
                    xs_hbm.at[sstg_v.at[stage % 2, r]],
                    rows_v.at[slot], gsem[slot])
            i = j - LOOK
            if i >= 0:
                stage_i, r_i = divmod(i, STG)
                gdesc.pop(i).wait()
                sdesc[i] = pltpu.async_copy(
                    rows_v.at[i % SLOTS],
                    acc_sh.at[dstg_v.at[stage_i % 2, r_i]],
                    ssem[i % SLOTS], add=True)

        for i in sorted(sdesc):
            sdesc[i].wait()

        plsc.subcore_barrier()
        pltpu.sync_copy(acc_sh.at[pl.ds(s * RPT, RPT)],
                        out_hbm.at[c, pl.ds(s * RPT, RPT)])

    return spmm


_spmm_edges = _make_spmm(False)
_spmm_feats = _make_spmm(True)


GB = 16
NGRP = N // GB
GPW = 20
WROWS = GPW * GB
GH = G * H


GID_PAD = 1 << 20


@functools.partial(
    pl.kernel,
    out_type=jax.ShapeDtypeStruct((NW, GH), jnp.float32),
    mesh=_mesh,
    scratch_types=[
        pltpu.VMEM((WROWS,), jnp.int32),
        pltpu.VMEM((GB, H), jnp.float32),
        pltpu.VMEM((GB, H), jnp.float32),
        pltpu.VMEM((GH,), jnp.float32),
        pltpu.SemaphoreType.DMA,
        pltpu.SemaphoreType.DMA,
    ],
)
def _segmax(x_hbm, gid_hbm, out_hbm, gid_v, rows_a, rows_b, acc_v,
            sem_a, sem_b):
    c = lax.axis_index("c")
    s = lax.axis_index("s")
    w = s * NC + c
    ninf = jnp.full((LANES,), -jnp.inf, jnp.float32)

    @pl.loop(0, GH // LANES)
    def _(i):
        acc_v[pl.ds(i * LANES, LANES)] = ninf

    pltpu.sync_copy(gid_hbm.at[pl.ds(w * WROWS, WROWS)], gid_v)
    ngroups = jnp.minimum(GPW, NGRP - w * GPW)
    npairs = (ngroups + 1) // 2

    def flush(prev_gid, regs):
        @pl.when((prev_gid >= 0) & (prev_gid < G))
        def _():
            for ch in range(H // LANES):
                acc_v[pl.ds(prev_gid * H + ch * LANES, LANES)] = regs[ch]

    def run_group(g, rows_v, carry):
        prev_gid = carry[0]
        regs = list(carry[1:])
        gv = gid_v[pl.ds(g * GB, GB)]
        for jj in range(GB):
            gid_j = gv[jj]
            changed = gid_j != prev_gid
            flush_now = changed & (prev_gid >= 0) & (prev_gid < G)

            @pl.when(flush_now)
            def _(prev_gid=prev_gid, regs=tuple(regs)):
                for ch in range(H // LANES):
                    acc_v[pl.ds(prev_gid * H + ch * LANES, LANES)] = regs[ch]

            regs = [
                jnp.maximum(jnp.where(changed, ninf, r),
                            rows_v[jj, pl.ds(ch * LANES, LANES)])
                for ch, r in enumerate(regs)
            ]
            prev_gid = gid_j
        return (prev_gid, *regs)

    def pair_body(p, carry):
        g0 = 2 * p
        r0 = w * WROWS + g0 * GB
        d0 = pltpu.async_copy(x_hbm.at[pl.ds(r0, GB)], rows_a, sem_a)
        have2 = g0 + 1 < ngroups

        @pl.when(have2)
        def _():
            pltpu.async_copy(x_hbm.at[pl.ds(r0 + GB, GB)], rows_b, sem_b)

        d0.wait()
        carry = run_group(g0, rows_a, carry)

        @pl.when(have2)
        def _():
            pltpu.make_async_copy(x_hbm.at[pl.ds(r0 + GB, GB)], rows_b,
                                  sem_b).wait()

        carry2 = run_group(g0 + 1, rows_b, carry)
        return tuple(
            jnp.where(have2, c2, c1) for c1, c2 in zip(carry, carry2))

    final = lax.fori_loop(0, npairs, pair_body,
                          (jnp.int32(-1), *([ninf] * (H // LANES))))
    flush(final[0], final[1:])
    pltpu.sync_copy(acc_v, out_hbm.at[w])


BN = 2000


def _prep_body(degp_ref, h_ref, nt_ref, xs_ref):
    d = jnp.sum(degp_ref[...].astype(jnp.float32), axis=1)
    nm = jnp.where(d > 0, lax.rsqrt(jnp.maximum(d, 1.0)), 0.0)
    nt = nm.T
    nt_ref[...] = nt
    xs_ref[...] = h_ref[...] * nt[:, 0:1]


_prep = pl.pallas_call(
    _prep_body,
    out_shape=[
        jax.ShapeDtypeStruct((N, 2), jnp.float32),
        jax.ShapeDtypeStruct((N, D), jnp.float32),
    ],
)


def _make_layer(din, scale_out):
    sum_parts = din == 128

    def body(*refs):
        if scale_out:
            agg_ref, ndst_ref, w_ref, b_ref, nsrc_ref, out_ref = refs
        else:
            agg_ref, ndst_ref, w_ref, b_ref, out_ref = refs
        if sum_parts:
            x = agg_ref[0] + agg_ref[1]
        else:
            x = jnp.concatenate([agg_ref[0], agg_ref[1]], axis=1)
        x = x * ndst_ref[...]
        y = jnp.dot(x, w_ref[...], preferred_element_type=jnp.float32)
        y = jnp.maximum(y + b_ref[...], 0.0)
        if scale_out:
            y = y * nsrc_ref[...]
            out_ref[0] = y[:, :H // 2]
            out_ref[1] = y[:, H // 2:]
        else:
            out_ref[...] = y

    in_specs = [
        pl.BlockSpec((2, BN, 128), lambda i: (0, i, 0)),
        pl.BlockSpec((BN, 1), lambda i: (i, 0)),
        pl.BlockSpec((din, H), lambda i: (0, 0)),
        pl.BlockSpec((1, H), lambda i: (0, 0)),
    ]
    if scale_out:
        in_specs.append(pl.BlockSpec((BN, 1), lambda i: (i, 0)))
        out_spec = pl.BlockSpec((2, BN, H // 2), lambda i: (0, i, 0))
        out_shape = jax.ShapeDtypeStruct((2, N, H // 2), jnp.float32)
    else:
        out_spec = pl.BlockSpec((BN, H), lambda i: (i, 0))
        out_shape = jax.ShapeDtypeStruct((N, H), jnp.float32)

    return pl.pallas_call(
        body,
        grid=(N // BN,),
        in_specs=in_specs,
        out_specs=out_spec,
        out_shape=out_shape,
    )


_layer1 = _make_layer(D, True)
_layer2 = _make_layer(H, False)


def _combine_body(p_ref, wc_ref, bc_ref, out_ref):
    hg = jnp.max(p_ref[...], axis=0)
    out_ref[...] = (
        jnp.dot(hg, wc_ref[...], preferred_element_type=jnp.float32)
        + bc_ref[...])


_combine = pl.pallas_call(
    _combine_body,
    out_shape=jax.ShapeDtypeStruct((G, C), jnp.float32),
)


def kernel(h, edge_index, graph_ids, W1, b1, W2, b2, Wc, bc):
    src = edge_index[0]
    dst = edge_index[1]

    degp = _degrees(src, dst)
    nt, xs1 = _prep(degp, h)
    nsrc = nt[:, 0:1]
    ndst = nt[:, 1:2]

    epad = jnp.arange(PADE, dtype=jnp.int32)
    src_p = jnp.concatenate([src, epad % N])
    dst_p = jnp.concatenate([dst, epad % (NPAD - N) + N])
    src2 = jnp.stack([src_p, src_p + N]).reshape(2, E2 // EK, EK)
    dst2 = dst_p.reshape(E2 // EK, EK)
    agg1 = _spmm_edges(xs1, src2, dst2)
    xs2 = _layer1(agg1, ndst, W1, b1.reshape(1, H), nsrc)
    agg2 = _spmm_feats(xs2.reshape(2 * N, H // 2), src2, dst2)
    x2 = _layer2(agg2, ndst, W2, b2.reshape(1, H))

    gpad = jnp.concatenate(
        [graph_ids.astype(jnp.int32),
         jnp.full((NPAD - N,), GID_PAD, jnp.int32)])
    parts = _segmax(x2, gpad)
    return _combine(parts.reshape(NW, G, H), Wc, bc.reshape(1, C))
